# baseline (speedup 1.0000x reference)
import functools

import jax
import jax.numpy as jnp
import numpy as np
from jax import lax
from jax.experimental import pallas as pl
from jax.experimental.pallas import tpu as pltpu
from jax.experimental.pallas import tpu_sc as plsc

_VOCAB = 1000000
_DIM = 64
_B = 4096
_L = 50
_KEEP_P = 0.75

_NC, _NS, _LANES = 2, 16, 16
_NW = _NC * _NS
_NROWS = _B * _L
_NSPLIT = 2
_NROWS_H = _NROWS // _NSPLIT


def _threefry2x32_np(k1, k2, x0, x1):
    def rotl(x, d):
        return (x << np.uint32(d)) | (x >> np.uint32(32 - d))

    rot0 = (13, 15, 26, 6)
    rot1 = (17, 29, 16, 24)
    ks0 = np.uint32(k1)
    ks1 = np.uint32(k2)
    ks2 = ks0 ^ ks1 ^ np.uint32(0x1BD11BDA)

    def rounds(x0, x1, rots):
        for r in rots:
            x0 = (x0 + x1).astype(np.uint32)
            x1 = x0 ^ rotl(x1, r)
        return x0, x1

    x0 = (x0 + ks0).astype(np.uint32)
    x1 = (x1 + ks1).astype(np.uint32)
    for i, (rots, ka, kb) in enumerate((
            (rot0, ks1, ks2), (rot1, ks2, ks0), (rot0, ks0, ks1),
            (rot1, ks1, ks2), (rot0, ks2, ks0))):
        x0, x1 = rounds(x0, x1, rots)
        x0 = (x0 + ka).astype(np.uint32)
        x1 = (x1 + kb + np.uint32(i + 1)).astype(np.uint32)
    return x0, x1


def _packed_keep_bits():
    n = _B * _L * _DIM
    i = np.arange(n, dtype=np.uint32)
    o0, o1 = _threefry2x32_np(0, 42, np.zeros(n, np.uint32), i)
    keep = ((o0 ^ o1) < np.uint32(0xC0000000))
    kw = keep.reshape(n // 512, 32, _LANES).astype(np.uint64)
    words = (kw << np.arange(32, dtype=np.uint64)[None, :, None]).sum(axis=1)
    return words.astype(np.uint32).reshape(-1)


_BITS = _packed_keep_bits()


def _make_sc_embed(nrows, chunk):
    rpw = nrows // _NW
    nt = rpw // chunk
    qpc = chunk * _DIM // (32 * _LANES)
    wpw = rpw * _DIM // 32
    assert nt % 2 == 0 and rpw % chunk == 0 and chunk % _LANES == 0

    def body(w_hbm, idx_hbm, bits_hbm, out_hbm,
             idx_all, bits_all, rows0, rows1, sem0, sem1):
        rows_bufs = (rows0, rows1)
        sems = (sem0, sem1)
        wid = lax.axis_index("s") * _NC + lax.axis_index("c")
        base_row = wid * rpw

        pltpu.sync_copy(idx_hbm.at[pl.ds(base_row, rpw)], idx_all)
        pltpu.sync_copy(bits_hbm.at[pl.ds(wid * wpw, wpw)], bits_all)

        def start_gather(t, buf):
            def group_body(gi, carry):
                v = idx_all[pl.ds(t * chunk + gi * _LANES, _LANES)]
                for j in range(_LANES):
                    pltpu.async_copy(
                        w_hbm.at[pl.ds(v[j], 1), :],
                        rows_bufs[buf].at[pl.ds(gi * _LANES + j, 1), :],
                        sems[buf])
                return carry

            lax.fori_loop(0, chunk // _LANES, group_body, 0)

        def wait_gather(buf):
            pltpu.make_async_copy(
                out_hbm.at[pl.ds(0, chunk), :], rows_bufs[buf],
                sems[buf]).wait()

        def apply_mask(t, buf):
            rows = rows_bufs[buf]

            def q_body(q, carry):
                w16 = bits_all[pl.ds((t * qpc + q) * _LANES, _LANES)]
                for g in range(32):
                    bit = (w16 >> np.uint32(g)) & np.uint32(1)
                    f = bit.astype(jnp.float32) * np.float32(1.0 / _KEEP_P)
                    r = 8 * q + (g // 4)
                    sl = 16 * (g % 4)
                    rows[r, sl:sl + 16] = rows[r, sl:sl + 16] * f
                return carry

            lax.fori_loop(0, qpc, q_body, 0)

        start_gather(0, 0)

        def t2_body(t2, carry):
            for b in range(2):
                t = 2 * t2 + b
                nxt = t + 1

                @pl.when(nxt < nt)
                def _():
                    start_gather(nxt, (b + 1) % 2)

                wait_gather(b)
                apply_mask(t, b)
                pltpu.sync_copy(
                    rows_bufs[b],
                    out_hbm.at[pl.ds(base_row + t * chunk, chunk), :])
            return carry

        lax.fori_loop(0, nt // 2, t2_body, 0)

    mesh = plsc.VectorSubcoreMesh(
        core_axis_name="c", subcore_axis_name="s",
        num_cores=_NC, num_subcores=_NS)
    return pl.kernel(
        body,
        out_type=jax.ShapeDtypeStruct((nrows, _DIM), jnp.float32),
        mesh=mesh,
        scratch_types=[
            pltpu.VMEM((rpw,), jnp.int32),
            pltpu.VMEM((wpw,), jnp.uint32),
            pltpu.VMEM((chunk, _DIM), jnp.float32),
            pltpu.VMEM((chunk, _DIM), jnp.float32),
            pltpu.SemaphoreType.DMA,
            pltpu.SemaphoreType.DMA,
        ],
    )


@jax.jit
def _embed(W, idx, bits):
    half = _make_sc_embed(_NROWS_H, 64)
    outs = []
    for s in range(_NSPLIT):
        r = half(W, idx[s * _NROWS_H:(s + 1) * _NROWS_H],
                 bits[s * _NROWS_H * 2:(s + 1) * _NROWS_H * 2])
        outs.append(r.reshape(_B // _NSPLIT, _L, _DIM))
    return jnp.concatenate(outs, axis=0)


def kernel(x, W):
    idx = x.reshape(_NROWS).astype(jnp.int32)
    return _embed(W, idx, jnp.asarray(_BITS))


# speedup vs baseline: 1.0176x; 1.0176x over previous
import functools

import jax
import jax.numpy as jnp
import numpy as np
from jax import lax
from jax.experimental import pallas as pl
from jax.experimental.pallas import tpu as pltpu
from jax.experimental.pallas import tpu_sc as plsc

_VOCAB = 1000000
_DIM = 64
_B = 4096
_L = 50
_KEEP_P = 0.75

_NC, _NS, _LANES = 2, 16, 16
_NW = _NC * _NS
_NROWS = _B * _L
_NSPLIT = 1
_NROWS_H = _NROWS // _NSPLIT


def _threefry2x32_np(k1, k2, x0, x1):
    def rotl(x, d):
        return (x << np.uint32(d)) | (x >> np.uint32(32 - d))

    rot0 = (13, 15, 26, 6)
    rot1 = (17, 29, 16, 24)
    ks0 = np.uint32(k1)
    ks1 = np.uint32(k2)
    ks2 = ks0 ^ ks1 ^ np.uint32(0x1BD11BDA)

    def rounds(x0, x1, rots):
        for r in rots:
            x0 = (x0 + x1).astype(np.uint32)
            x1 = x0 ^ rotl(x1, r)
        return x0, x1

    x0 = (x0 + ks0).astype(np.uint32)
    x1 = (x1 + ks1).astype(np.uint32)
    for i, (rots, ka, kb) in enumerate((
            (rot0, ks1, ks2), (rot1, ks2, ks0), (rot0, ks0, ks1),
            (rot1, ks1, ks2), (rot0, ks2, ks0))):
        x0, x1 = rounds(x0, x1, rots)
        x0 = (x0 + ka).astype(np.uint32)
        x1 = (x1 + kb + np.uint32(i + 1)).astype(np.uint32)
    return x0, x1


def _packed_keep_bits():
    n = _B * _L * _DIM
    i = np.arange(n, dtype=np.uint32)
    o0, o1 = _threefry2x32_np(0, 42, np.zeros(n, np.uint32), i)
    keep = ((o0 ^ o1) < np.uint32(0xC0000000))
    kw = keep.reshape(n // 512, 32, _LANES).astype(np.uint64)
    words = (kw << np.arange(32, dtype=np.uint64)[None, :, None]).sum(axis=1)
    return words.astype(np.uint32).reshape(-1)


_BITS = _packed_keep_bits()


def _make_sc_embed(nrows, chunk):
    rpw = nrows // _NW
    nt = rpw // chunk
    qpc = chunk * _DIM // (32 * _LANES)
    wpw = rpw * _DIM // 32
    assert nt % 2 == 0 and rpw % chunk == 0 and chunk % _LANES == 0

    def body(w_hbm, idx_hbm, bits_hbm, out_hbm,
             idx_all, bits_all, rows0, rows1, sem0, sem1):
        rows_bufs = (rows0, rows1)
        sems = (sem0, sem1)
        wid = lax.axis_index("s") * _NC + lax.axis_index("c")
        base_row = wid * rpw

        pltpu.sync_copy(idx_hbm.at[pl.ds(base_row, rpw)], idx_all)
        pltpu.sync_copy(bits_hbm.at[pl.ds(wid * wpw, wpw)], bits_all)

        def start_gather(t, buf):
            def group_body(gi, carry):
                v = idx_all[pl.ds(t * chunk + gi * _LANES, _LANES)]
                for j in range(_LANES):
                    pltpu.async_copy(
                        w_hbm.at[pl.ds(v[j], 1), :],
                        rows_bufs[buf].at[pl.ds(gi * _LANES + j, 1), :],
                        sems[buf])
                return carry

            lax.fori_loop(0, chunk // _LANES, group_body, 0)

        def wait_gather(buf):
            pltpu.make_async_copy(
                out_hbm.at[pl.ds(0, chunk), :], rows_bufs[buf],
                sems[buf]).wait()

        def apply_mask(t, buf):
            rows = rows_bufs[buf]

            def q_body(q, carry):
                w16 = bits_all[pl.ds((t * qpc + q) * _LANES, _LANES)]
                for g in range(32):
                    bit = (w16 >> np.uint32(g)) & np.uint32(1)
                    f = bit.astype(jnp.float32) * np.float32(1.0 / _KEEP_P)
                    r = 8 * q + (g // 4)
                    sl = 16 * (g % 4)
                    rows[r, sl:sl + 16] = rows[r, sl:sl + 16] * f
                return carry

            lax.fori_loop(0, qpc, q_body, 0)

        start_gather(0, 0)

        def t2_body(t2, carry):
            for b in range(2):
                t = 2 * t2 + b
                nxt = t + 1

                @pl.when(nxt < nt)
                def _():
                    start_gather(nxt, (b + 1) % 2)

                wait_gather(b)
                apply_mask(t, b)
                pltpu.sync_copy(
                    rows_bufs[b],
                    out_hbm.at[pl.ds(base_row + t * chunk, chunk), :])
            return carry

        lax.fori_loop(0, nt // 2, t2_body, 0)

    mesh = plsc.VectorSubcoreMesh(
        core_axis_name="c", subcore_axis_name="s",
        num_cores=_NC, num_subcores=_NS)
    return pl.kernel(
        body,
        out_type=jax.ShapeDtypeStruct((nrows, _DIM), jnp.float32),
        mesh=mesh,
        scratch_types=[
            pltpu.VMEM((rpw,), jnp.int32),
            pltpu.VMEM((wpw,), jnp.uint32),
            pltpu.VMEM((chunk, _DIM), jnp.float32),
            pltpu.VMEM((chunk, _DIM), jnp.float32),
            pltpu.SemaphoreType.DMA,
            pltpu.SemaphoreType.DMA,
        ],
    )


@jax.jit
def _embed(W, idx, bits):
    part = _make_sc_embed(_NROWS_H, 128)
    outs = []
    for s in range(_NSPLIT):
        r = part(W, idx[s * _NROWS_H:(s + 1) * _NROWS_H],
                 bits[s * _NROWS_H * 2:(s + 1) * _NROWS_H * 2])
        outs.append(r.reshape(_B // _NSPLIT, _L, _DIM))
    if _NSPLIT == 1:
        return outs[0]
    return jnp.concatenate(outs, axis=0)


def kernel(x, W):
    idx = x.reshape(_NROWS).astype(jnp.int32)
    return _embed(W, idx, jnp.asarray(_BITS))
